# Initial kernel scaffold; baseline (speedup 1.0000x reference)
import jax
import jax.numpy as jnp
from jax.experimental import pallas as pl


def kernel(features, edge_index, W0, b0, W1, b1, W2, b2):
    raise NotImplementedError("write your pallas kernel here")


# speedup vs baseline: 4.9268x; 4.9268x over previous
import jax
import jax.numpy as jnp
from jax import lax
from jax.experimental import pallas as pl
from jax.experimental.pallas import tpu as pltpu
from jax.experimental.pallas import tpu_sc as plsc

N = 10000
E = 320000
D = 128
NP = 10240
NC = 2
NS = 16
NW = NC * NS
EW = E // NW
BK = 80
NB = EW // BK
STRIPE = NP // NS

_mesh = plsc.VectorSubcoreMesh(core_axis_name="c", subcore_axis_name="s")


def _agg_body(y, eidx3, eidx4, zeros, aggp, srcv, dstv, buf0, agg_sp, sem0):
    c = lax.axis_index("c")
    s = lax.axis_index("s")
    wid = c * NS + s
    rows = pl.ds(s * STRIPE, STRIPE)
    pltpu.sync_copy(zeros.at[rows], agg_sp.at[rows])
    pltpu.sync_copy(eidx3.at[0, wid], srcv)
    pltpu.sync_copy(eidx4.at[1, wid], dstv)
    plsc.subcore_barrier()

    def body(j, carry):
        pltpu.async_copy(y.at[srcv.at[pl.ds(j * BK, BK)]], buf0, sem0).wait()
        pltpu.sync_copy(buf0, agg_sp.at[dstv.at[j]], add=True)
        return carry

    lax.fori_loop(0, NB, body, 0)
    plsc.subcore_barrier()
    pltpu.sync_copy(agg_sp.at[rows], aggp.at[c, rows])


_sc_agg = pl.kernel(
    _agg_body,
    out_type=jax.ShapeDtypeStruct((NC, NP, D), jnp.float32),
    mesh=_mesh,
    scratch_types=[
        pltpu.VMEM((EW,), jnp.int32),
        pltpu.VMEM((NB, BK), jnp.int32),
        pltpu.VMEM((BK, D), jnp.float32),
        pltpu.VMEM_SHARED((NP, D), jnp.float32),
        pltpu.SemaphoreType.DMA,
    ],
)


BN = 1024


def _first_body(degpo_ref, degpi_ref, feat_ref, w_ref, y_ref, sout_ref, sin_ref):
    dout = jnp.max(degpo_ref[0] + degpo_ref[1], axis=-1, keepdims=True)
    din = jnp.max(degpi_ref[0] + degpi_ref[1], axis=-1, keepdims=True)
    so = lax.rsqrt(jnp.maximum(dout, 1.0))
    si = lax.rsqrt(jnp.maximum(din, 1.0))
    sout_ref[...] = so
    sin_ref[...] = si
    y_ref[...] = jnp.dot(feat_ref[...] * so, w_ref[...],
                         preferred_element_type=jnp.float32)


def _tc_first(degpo, degpi, featp, w):
    return pl.pallas_call(
        _first_body,
        grid=(NP // BN,),
        in_specs=[
            pl.BlockSpec((NC, BN, D), lambda i: (0, i, 0)),
            pl.BlockSpec((NC, BN, D), lambda i: (0, i, 0)),
            pl.BlockSpec((BN, D), lambda i: (i, 0)),
            pl.BlockSpec((D, D), lambda i: (0, 0)),
        ],
        out_specs=[
            pl.BlockSpec((BN, D), lambda i: (i, 0)),
            pl.BlockSpec((BN, 1), lambda i: (i, 0)),
            pl.BlockSpec((BN, 1), lambda i: (i, 0)),
        ],
        out_shape=[
            jax.ShapeDtypeStruct((NP, D), jnp.float32),
            jax.ShapeDtypeStruct((NP, 1), jnp.float32),
            jax.ShapeDtypeStruct((NP, 1), jnp.float32),
        ],
    )(degpo, degpi, featp, w)


def _mid_body(aggp_ref, sin_ref, b_ref, sout_ref, w_ref, y_ref):
    a = aggp_ref[0] + aggp_ref[1]
    h = jnp.maximum(a * sin_ref[...] + b_ref[...], 0.0)
    y_ref[...] = jnp.dot(h * sout_ref[...], w_ref[...],
                         preferred_element_type=jnp.float32)


def _tc_mid(aggp, sin, b, sout, w):
    return pl.pallas_call(
        _mid_body,
        grid=(NP // BN,),
        in_specs=[
            pl.BlockSpec((NC, BN, D), lambda i: (0, i, 0)),
            pl.BlockSpec((BN, 1), lambda i: (i, 0)),
            pl.BlockSpec((1, D), lambda i: (0, 0)),
            pl.BlockSpec((BN, 1), lambda i: (i, 0)),
            pl.BlockSpec((D, D), lambda i: (0, 0)),
        ],
        out_specs=pl.BlockSpec((BN, D), lambda i: (i, 0)),
        out_shape=jax.ShapeDtypeStruct((NP, D), jnp.float32),
    )(aggp, sin, b, sout, w)


def _fin_body(aggp_ref, sin_ref, b_ref, out_ref):
    a = aggp_ref[0] + aggp_ref[1]
    out_ref[...] = a * sin_ref[...] + b_ref[...]


def _tc_fin(aggp, sin, b):
    return pl.pallas_call(
        _fin_body,
        grid=(NP // BN,),
        in_specs=[
            pl.BlockSpec((NC, BN, D), lambda i: (0, i, 0)),
            pl.BlockSpec((BN, 1), lambda i: (i, 0)),
            pl.BlockSpec((1, D), lambda i: (0, 0)),
        ],
        out_specs=pl.BlockSpec((BN, D), lambda i: (i, 0)),
        out_shape=jax.ShapeDtypeStruct((NP, D), jnp.float32),
    )(aggp, sin, b)


@jax.jit
def kernel(features, edge_index, W0, b0, W1, b1, W2, b2):
    eidx3 = edge_index.reshape(2, NW, EW)
    eidx4 = edge_index.reshape(2, NW, NB, BK)
    ei_sw = edge_index[::-1]
    eidx3s = ei_sw.reshape(2, NW, EW)
    eidx4s = ei_sw.reshape(2, NW, NB, BK)
    zeros = jnp.zeros((NP, D), jnp.float32)
    ones = jnp.ones((NP, D), jnp.float32)
    featp = jnp.pad(features, ((0, NP - N), (0, 0)))

    degpi = _sc_agg(ones, eidx3, eidx4, zeros)
    degpo = _sc_agg(ones, eidx3s, eidx4s, zeros)
    y0, sout, sin = _tc_first(degpo, degpi, featp, W0)
    aggp = _sc_agg(y0, eidx3, eidx4, zeros)
    y1 = _tc_mid(aggp, sin, b0.reshape(1, D), sout, W1)
    aggp = _sc_agg(y1, eidx3, eidx4, zeros)
    y2 = _tc_mid(aggp, sin, b1.reshape(1, D), sout, W2)
    aggp = _sc_agg(y2, eidx3, eidx4, zeros)
    out = _tc_fin(aggp, sin, b2.reshape(1, D))
    return out[:N]
